# baseline (speedup 1.0000x reference)
import functools

import jax
import jax.numpy as jnp
from jax import lax
from jax.experimental import pallas as pl
from jax.experimental.pallas import tpu as pltpu
from jax.experimental.pallas import tpu_sc as plsc

NT = 1024
NH = 512
QCHUNK = 32


def _nn_half(u, kk):
    m = kk.shape[1]
    un = jnp.sum(u * u, axis=1, keepdims=True)
    e = (-2.0 * u[:, 0])[:, None] * kk[0][None, :]
    e = e + (-2.0 * u[:, 1])[:, None] * kk[1][None, :]
    e = e + ((-2.0 * u[:, 2])[:, None] * kk[2][None, :] + kk[3][None, :])

    inf = jnp.float32(jnp.inf)
    colid = jax.lax.broadcasted_iota(jnp.int32, (NH, m), 1)
    idxs = []
    recips = []
    for t in range(3):
        mn = jnp.min(e, axis=1, keepdims=True)
        sel = e == mn
        amin = jnp.min(jnp.where(sel, colid, m), axis=1, keepdims=True)
        if t < 2:
            e = jnp.where(sel, inf, e)
        idxs.append(amin)
        recips.append(jax.lax.rsqrt(jnp.maximum(mn + un, 1e-24)))

    wsum = recips[0] + recips[1] + recips[2]
    w = jnp.concatenate(
        recips + [jnp.zeros((NH, 13), jnp.float32)], axis=1
    ) / wsum
    gi = jnp.concatenate(idxs, axis=1)
    return gi, w


def _nn_body(u_ref, k_ref, idx_ref, w_ref):
    b = pl.program_id(0)
    kk = k_ref[0]
    m = kk.shape[1]
    for h in range(2):
        gi, w = _nn_half(u_ref[0, h * NH:(h + 1) * NH], kk)
        idx_ref[0, h * NH:(h + 1) * NH] = gi + b * m
        w_ref[0, h * NH:(h + 1) * NH] = w


def _nn(unknown, known):
    B, n, _ = unknown.shape
    _, m, _ = known.shape
    kt = known.transpose(0, 2, 1)
    kn = jnp.sum(known * known, axis=2)[:, None, :]
    k = jnp.concatenate([kt, kn, jnp.zeros((B, 4, m), jnp.float32)], axis=1)
    return pl.pallas_call(
        _nn_body,
        grid=(B, n // NT),
        in_specs=[
            pl.BlockSpec((1, NT, 3), lambda b, i: (b, i, 0)),
            pl.BlockSpec((1, 8, m), lambda b, i: (b, 0, 0)),
        ],
        out_specs=[
            pl.BlockSpec((1, NT, 3), lambda b, i: (b, i, 0)),
            pl.BlockSpec((1, NT, 16), lambda b, i: (b, i, 0)),
        ],
        out_shape=[
            jax.ShapeDtypeStruct((B, n, 3), jnp.int32),
            jax.ShapeDtypeStruct((B, n, 16), jnp.float32),
        ],
    )(unknown, k)


def _interp(table, idx_flat, w_flat, total_q):
    C = table.shape[1]
    info = plsc.get_sparse_core_info()
    nw = info.num_cores * info.num_subcores
    qpw = total_q // nw
    nchunks = qpw // QCHUNK
    mesh = plsc.VectorSubcoreMesh(core_axis_name="c", subcore_axis_name="s")

    @functools.partial(
        pl.kernel,
        mesh=mesh,
        out_type=jax.ShapeDtypeStruct((total_q, C), jnp.float32),
        scratch_types=[
            pltpu.VMEM((3 * QCHUNK,), jnp.int32),
            pltpu.VMEM((16 * QCHUNK,), jnp.float32),
            pltpu.VMEM((3 * QCHUNK, C), jnp.float32),
            pltpu.VMEM((QCHUNK, C), jnp.float32),
            pltpu.SemaphoreType.DMA,
        ],
    )
    def k(table_hbm, idx_hbm, w_hbm, out_hbm, idx_v, w_v, rows_v, out_v, sem):
        wid = lax.axis_index("s") * info.num_cores + lax.axis_index("c")
        qbase = wid * qpw

        def chunk_body(ci, _):
            q0 = qbase + ci * QCHUNK
            pltpu.sync_copy(idx_hbm.at[pl.ds(q0 * 3, 3 * QCHUNK)], idx_v)
            pltpu.sync_copy(w_hbm.at[pl.ds(q0 * 16, 16 * QCHUNK)], w_v)
            pltpu.async_copy(table_hbm.at[idx_v], rows_v, sem).wait()

            def q_body(q, _):
                wvec = w_v[pl.ds(q * 16, 16)]
                w0 = wvec[0]
                w1 = wvec[1]
                w2 = wvec[2]
                for j in range(C // 16):
                    sl = pl.ds(j * 16, 16)
                    acc = (
                        w0 * rows_v[3 * q, sl]
                        + w1 * rows_v[3 * q + 1, sl]
                        + w2 * rows_v[3 * q + 2, sl]
                    )
                    out_v[q, sl] = acc
                return 0

            lax.fori_loop(0, QCHUNK, q_body, 0)
            pltpu.sync_copy(out_v, out_hbm.at[pl.ds(q0, QCHUNK)])
            return 0

        lax.fori_loop(0, nchunks, chunk_body, 0)

    return k(table, idx_flat, w_flat)


def kernel(unknown, known, known_feats):
    B, n, _ = unknown.shape
    _, m, _ = known.shape
    C = known_feats.shape[1]

    idx, w = _nn(unknown, known)
    table = known_feats.transpose(0, 2, 1).reshape(B * m, C)
    out_t = _interp(table, idx.reshape(-1), w.reshape(-1), B * n)
    return out_t.reshape(B, n, C).transpose(0, 2, 1)


# speedup vs baseline: 3.4509x; 3.4509x over previous
import jax
import jax.numpy as jnp
from jax.experimental import pallas as pl

NT = 2048
NH = 512
NC = NT // NH


def _chain(u, kk, f):
    m = kk.shape[1]
    un = jnp.sum(u * u, axis=1, keepdims=True)
    e = (-2.0 * u[:, 0])[:, None] * kk[0][None, :]
    e = e + (-2.0 * u[:, 1])[:, None] * kk[1][None, :]
    e = e + ((-2.0 * u[:, 2])[:, None] * kk[2][None, :] + kk[3][None, :])

    inf = jnp.float32(jnp.inf)
    wt = jnp.zeros((NH, m), jnp.float32)
    wsum = jnp.zeros((NH, 1), jnp.float32)
    for t in range(3):
        mn = jnp.min(e, axis=1, keepdims=True)
        sel = e == mn
        if t < 2:
            e = jnp.where(sel, inf, e)
        recip_t = jax.lax.rsqrt(jnp.maximum(mn + un, 1e-24))
        wt = jnp.where(sel, recip_t, wt)
        wsum = wsum + recip_t

    out = jax.lax.dot_general(
        f, wt, (((1,), (1,)), ((), ())),
        preferred_element_type=jnp.float32,
    )
    return out * (1.0 / wsum).reshape(1, NH)


def _body(u_ref, k_ref, f_ref, o_ref):
    kk = k_ref[0]
    f = f_ref[0]
    for h in range(NC):
        out = _chain(u_ref[0, h * NH:(h + 1) * NH], kk, f)
        o_ref[0, :, h * NH:(h + 1) * NH] = out


def kernel(unknown, known, known_feats):
    B, n, _ = unknown.shape
    _, m, _ = known.shape
    C = known_feats.shape[1]

    kt = known.transpose(0, 2, 1)
    kn = jnp.sum(known * known, axis=2)[:, None, :]
    k = jnp.concatenate(
        [kt, kn, jnp.zeros((B, 4, m), jnp.float32)], axis=1
    )

    return pl.pallas_call(
        _body,
        grid=(B, n // NT),
        in_specs=[
            pl.BlockSpec((1, NT, 3), lambda b, i: (b, i, 0)),
            pl.BlockSpec((1, 8, m), lambda b, i: (b, 0, 0)),
            pl.BlockSpec((1, C, m), lambda b, i: (b, 0, 0)),
        ],
        out_specs=pl.BlockSpec((1, C, NT), lambda b, i: (b, 0, i)),
        out_shape=jax.ShapeDtypeStruct((B, C, n), jnp.float32),
    )(unknown, k, known_feats)
